# baseline (speedup 1.0000x reference)
import jax
import jax.numpy as jnp
from jax.experimental import pallas as pl


def _lin(z, W, b):
    return z @ W + b


def _mlp(z, p):
    return _lin(jax.nn.relu(_lin(z, p[0], p[1])), p[2], p[3])


def _scatter_stats(vals, idx, size):
    s = jax.ops.segment_sum(vals, idx, num_segments=size)
    m = jax.ops.segment_max(vals, idx, num_segments=size)
    m = jnp.where(jnp.isfinite(m), m, 0.0)
    cnt = jax.ops.segment_sum(jnp.ones((vals.shape[0], 1), vals.dtype), idx, num_segments=size)
    mean = s / jnp.maximum(cnt, 1.0)
    return s, m, mean


def _head_kernel(out_ref, w0, b0, w1, b1, w2, b2, w3, b3, o_ref):
    h = jnp.maximum(out_ref[...] @ w0[...] + b0[...], 0.0)
    h = jnp.maximum(h @ w1[...] + b1[...], 0.0)
    h = jnp.maximum(h @ w2[...] + b2[...], 0.0)
    o_ref[...] = h @ w3[...] + b3[...]


def kernel(x, edge_index, edge_attr, batch, weights):
    N = x.shape[0]
    row, col = edge_index[0], edge_index[1]
    w = weights
    ea = _mlp(edge_attr, w[0:4])
    s, m, mean = _scatter_stats(ea, col, N)
    x = _mlp(jnp.concatenate([s, m, mean], axis=1), w[4:8])
    for i in range(2):
        base = 8 + 8 * i
        ea = _mlp(jnp.concatenate([x[row], x[col], ea], axis=1), w[base:base + 4])
        s, m, mean = _scatter_stats(ea, col, N)
        x = _mlp(jnp.concatenate([x, s, m, mean], axis=1), w[base + 4:base + 8])
    gs, gm, gmean = _scatter_stats(x, batch, 64)
    out = jnp.concatenate([gs, gmean, gm], axis=1)
    ow = [jnp.asarray(a) for a in w[24:32]]
    return pl.pallas_call(
        _head_kernel,
        out_shape=jax.ShapeDtypeStruct((out.shape[0], ow[6].shape[1]), out.dtype),
    )(out, *ow)


# speedup vs baseline: 2.9266x; 2.9266x over previous
import functools

import jax
import jax.numpy as jnp
from jax import lax
from jax.experimental import pallas as pl
from jax.experimental.pallas import tpu as pltpu
from jax.experimental.pallas import tpu_sc as plsc

NN = 10000
NE = 320000
D = 64
NG = 64
NW = 32
E2 = NE // 2
R = 320

FL = 1024
LCAP = NE + FL + 16
CE = 8000
CHS = 512
CHG = 400

_MESH = plsc.VectorSubcoreMesh(core_axis_name="c", subcore_axis_name="s")
_CP_SC = pltpu.CompilerParams(use_tc_tiling_on_sc=False, needs_layout_passes=False)


def _wid():
    return lax.axis_index("s") * 2 + lax.axis_index("c")


@functools.partial(
    pl.kernel, mesh=_MESH, compiler_params=_CP_SC,
    out_type=(jax.ShapeDtypeStruct((NW, LCAP), jnp.int32),
              jax.ShapeDtypeStruct((NW, 8), jnp.int32)),
    scratch_types=[pltpu.VMEM((CE,), jnp.int32),
                   pltpu.VMEM((FL + 32,), jnp.int32),
                   pltpu.VMEM((16,), jnp.int32)],
)
def _sc_filter(col_hbm, list_hbm, cnt_hbm, col_v, buf_v, tmp_v):
    wid = _wid()
    lo = wid * R
    hi = jnp.minimum(lo + R, NN)
    lane = lax.broadcasted_iota(jnp.int32, (16,), 0)

    def chunk_body(ci, carry):
        pltpu.sync_copy(col_hbm.at[pl.ds(ci * CE, CE)], col_v)

        def body(jj, carry):
            off_loc, hbm_off = carry
            c = col_v[pl.ds(jj * 16, 16)]
            gid = ci * CE + jj * 16 + lane
            n = c - lo
            msk = jnp.logical_and(c >= lo, c < hi)
            glin = jnp.where(gid < E2, gid * 2, gid * 2 - (NE - 1))
            packed = glin + (n << 19)
            key = jnp.where(msk, lane, lane + 16)
            _, vals = plsc.sort_key_val(key, packed)
            buf_v[pl.ds(off_loc, 16)] = vals
            off_loc = off_loc + plsc.all_reduce_population_count(msk)[0]

            def flush(carry):
                off_loc, hbm_off = carry
                pltpu.sync_copy(buf_v.at[pl.ds(0, FL)],
                                list_hbm.at[wid, pl.ds(pl.multiple_of(hbm_off, FL), FL)])
                tail = buf_v[pl.ds(FL, 16)]
                buf_v[pl.ds(0, 16)] = tail
                return off_loc - FL, hbm_off + FL
            return lax.cond(off_loc >= FL, flush, lambda c: c, (off_loc, hbm_off))

        return lax.fori_loop(0, CE // 16, body, carry)

    off_loc, hbm_off = lax.fori_loop(0, NE // CE, chunk_body, (0, 0))
    pltpu.sync_copy(buf_v.at[pl.ds(0, FL)],
                    list_hbm.at[wid, pl.ds(pl.multiple_of(hbm_off, FL), FL)])
    tmp_v[...] = jnp.full((16,), hbm_off + off_loc, jnp.int32)
    pltpu.sync_copy(tmp_v.at[pl.ds(0, 8)], cnt_hbm.at[wid])


def _sc_scatter_body(with_cnt, *refs):
    if with_cnt:
        (list_hbm, cnts_hbm, ea_hbm, sp_hbm, m_hbm, c_hbm,
         pk_v, ids_v, cols_v, ea_v, macc_v, cnt_v, c8_v, acc_sh, sem) = refs
    else:
        (list_hbm, cnts_hbm, ea_hbm, sp_hbm, m_hbm,
         pk_v, ids_v, cols_v, ea_v, macc_v, cnt_v, c8_v, acc_sh, sem) = refs
    cid = lax.axis_index("c")
    sid = lax.axis_index("s")
    wid = sid * 2 + cid
    lo = wid * R
    lane = lax.broadcasted_iota(jnp.int32, (16,), 0)
    one0 = jnp.where(lane == 0, 1, 0)
    zero16 = jnp.zeros((16,), jnp.float32)
    ninf16 = jnp.full((16,), -jnp.inf, jnp.float32)

    def zb(i, _):
        for t in range(D // 16):
            macc_v[i, pl.ds(16 * t, 16)] = zero16
        return 0
    lax.fori_loop(0, 125, zb, 0)
    for k in range(5):
        pltpu.sync_copy(macc_v.at[pl.ds(0, 125)], acc_sh.at[pl.ds(sid * 625 + k * 125, 125)])
    @pl.when(sid == 15)
    def _():
        pltpu.sync_copy(macc_v.at[pl.ds(0, 16)], acc_sh.at[pl.ds(NN, 16)])

    def im(i, _):
        for t in range(D // 16):
            macc_v[i, pl.ds(16 * t, 16)] = ninf16
        return 0
    lax.fori_loop(0, R + 8, im, 0)
    if with_cnt:
        def ic(i, _):
            cnt_v[pl.ds(16 * i, 16)] = jnp.zeros((16,), jnp.int32)
            return 0
        lax.fori_loop(0, (R + 24) // 16, ic, 0)

    plsc.subcore_barrier()

    pltpu.sync_copy(cnts_hbm.at[wid], c8_v.at[pl.ds(0, 8)])
    count = c8_v[pl.ds(0, 16)][0]
    nch = (count + CHS - 1) // CHS

    def chunk(ci, _):
        base = ci * CHS
        pltpu.sync_copy(list_hbm.at[wid, pl.ds(pl.multiple_of(base, CHS), CHS)], pk_v)

        def up(jj, _):
            pk = pk_v[pl.ds(jj * 16, 16)]
            valid = (base + jj * 16 + lane) < count
            pkc = jnp.where(valid, pk, R << 19)
            pk_v[pl.ds(jj * 16, 16)] = pkc
            ids_v[pl.ds(jj * 16, 16)] = jnp.where(valid, pk & 0x7FFFF, 0)
            cols_v[pl.ds(jj * 16, 16)] = jnp.where(valid, (pk >> 19) + lo, NN)
            return 0
        lax.fori_loop(0, CHS // 16, up, 0)

        pltpu.async_copy(ea_hbm.at[ids_v], ea_v, sem).wait()
        pltpu.sync_copy(ea_v, acc_sh.at[cols_v], add=True)

        def g16(jj, _):
            pk16 = pk_v[pl.ds(jj * 16, 16)]
            for l in range(16):
                n = pk16[l] >> 19
                j = jj * 16 + l
                for t in range(D // 16):
                    v = ea_v[j, pl.ds(16 * t, 16)]
                    macc_v[n, pl.ds(16 * t, 16)] = jnp.maximum(macc_v[n, pl.ds(16 * t, 16)], v)
                if with_cnt:
                    cnt_v[pl.ds(n, 16)] += one0
            return 0
        lax.fori_loop(0, CHS // 16, g16, 0)
        return 0
    lax.fori_loop(0, nch, chunk, 0)

    @pl.when(wid < NW - 1)
    def _():
        pltpu.sync_copy(macc_v.at[pl.ds(0, R)], m_hbm.at[pl.ds(lo, R)])
        if with_cnt:
            pltpu.sync_copy(cnt_v.at[pl.ds(0, R)],
                            c_hbm.at[pl.ds(pl.multiple_of(lo, 8), R)])
    @pl.when(wid == NW - 1)
    def _():
        pltpu.sync_copy(macc_v.at[pl.ds(0, 80)], m_hbm.at[pl.ds(NN - 80, 80)])
        if with_cnt:
            pltpu.sync_copy(cnt_v.at[pl.ds(0, 80)], c_hbm.at[pl.ds(NN - 80, 80)])

    plsc.subcore_barrier()
    @pl.when(sid == 0)
    def _():
        pltpu.sync_copy(acc_sh.at[pl.ds(0, NN)], sp_hbm.at[cid])


def _make_scatter(with_cnt):
    outs = [jax.ShapeDtypeStruct((2, NN, D), jnp.float32),
            jax.ShapeDtypeStruct((NN, D), jnp.float32)]
    if with_cnt:
        outs.append(jax.ShapeDtypeStruct((NN,), jnp.int32))
    return functools.partial(
        pl.kernel, mesh=_MESH, compiler_params=_CP_SC,
        out_type=tuple(outs),
        scratch_types=[pltpu.VMEM((CHS,), jnp.int32),
                       pltpu.VMEM((CHS,), jnp.int32),
                       pltpu.VMEM((CHS,), jnp.int32),
                       pltpu.VMEM((CHS, D), jnp.float32),
                       pltpu.VMEM((R + 8, D), jnp.float32),
                       pltpu.VMEM((R + 24,), jnp.int32),
                       pltpu.VMEM((16,), jnp.int32),
                       pltpu.VMEM_SHARED((NN + 16, D), jnp.float32),
                       pltpu.SemaphoreType.DMA],
    )(functools.partial(_sc_scatter_body, with_cnt))


_sc_scatter_cnt = _make_scatter(True)
_sc_scatter = _make_scatter(False)


@functools.partial(
    pl.kernel, mesh=_MESH, compiler_params=_CP_SC,
    out_type=jax.ShapeDtypeStruct((NE, D), jnp.float32),
    scratch_types=[pltpu.VMEM((CHG,), jnp.int32),
                   pltpu.VMEM((CHG,), jnp.int32),
                   pltpu.VMEM((2 * CHG,), jnp.int32),
                   pltpu.VMEM((2 * CHG, D), jnp.float32),
                   pltpu.VMEM((CHG, D), jnp.float32),
                   pltpu.SemaphoreType.DMA],
)
def _sc_gather(row_hbm, col_hbm, ab_hbm, g_hbm, str_v, stc_v, gl_v, rows_v, g_v, sem):
    wid = _wid()
    lane = lax.broadcasted_iota(jnp.int32, (16,), 0)
    H = CHG // 2
    ebase = wid * (NE // NW) // 2

    def chunk(ci, _):
        base = wid * (NE // NW) + ci * CHG
        ea = ebase + ci * H
        pltpu.sync_copy(row_hbm.at[pl.ds(pl.multiple_of(ea, 8), H)], str_v.at[pl.ds(0, H)])
        pltpu.sync_copy(row_hbm.at[pl.ds(pl.multiple_of(E2 + ea, 8), H)], str_v.at[pl.ds(H, H)])
        pltpu.sync_copy(col_hbm.at[pl.ds(pl.multiple_of(ea, 8), H)], stc_v.at[pl.ds(0, H)])
        pltpu.sync_copy(col_hbm.at[pl.ds(pl.multiple_of(E2 + ea, 8), H)], stc_v.at[pl.ds(H, H)])

        def up(jj, _):
            lm = jj * 16 + lane
            src = (lm >> 1) + (lm & 1) * H
            gl_v[pl.ds(jj * 16, 16)] = plsc.load_gather(str_v, [src])
            gl_v[pl.ds(CHG + jj * 16, 16)] = plsc.load_gather(stc_v, [src]) + NN
            return 0
        lax.fori_loop(0, CHG // 16, up, 0)

        pltpu.async_copy(ab_hbm.at[gl_v], rows_v, sem).wait()

        def add(j, _):
            for t in range(D // 16):
                g_v[j, pl.ds(16 * t, 16)] = (rows_v[j, pl.ds(16 * t, 16)]
                                             + rows_v[CHG + j, pl.ds(16 * t, 16)])
            return 0
        lax.fori_loop(0, CHG, add, 0)
        pltpu.sync_copy(g_v, g_hbm.at[pl.ds(base, CHG)])
        return 0
    lax.fori_loop(0, (NE // NW) // CHG, chunk, 0)


EB = 4000
NB = 2000


def _t1_body(a_ref, b_ref, w0, b0, w1, b1, o_ref):
    ha = jnp.maximum(a_ref[...] @ w0[...] + b0[...], 0.0) @ w1[...] + b1[...]
    hb = jnp.maximum(b_ref[...] @ w0[...] + b0[...], 0.0) @ w1[...] + b1[...]
    o_ref[...] = jnp.concatenate([ha, hb], axis=1)


def _edge_mlp0(edge_attr, w0, b0, w1, b1):
    nb = E2 // EB
    return pl.pallas_call(
        _t1_body,
        grid=(nb,),
        in_specs=[pl.BlockSpec((EB, 16), lambda i: (i, 0)),
                  pl.BlockSpec((EB, 16), lambda i: (i + E2 // EB, 0)),
                  pl.BlockSpec((16, D), lambda i: (0, 0)),
                  pl.BlockSpec((1, D), lambda i: (0, 0)),
                  pl.BlockSpec((D, D), lambda i: (0, 0)),
                  pl.BlockSpec((1, D), lambda i: (0, 0))],
        out_specs=pl.BlockSpec((EB, 2 * D), lambda i: (i, 0)),
        out_shape=jax.ShapeDtypeStruct((E2, 2 * D), jnp.float32),
    )(edge_attr, edge_attr, w0, b0, w1, b1)


def _t2_body(g_ref, ea_ref, wh, w1, b1, o_ref):
    h = jnp.maximum(g_ref[...] + ea_ref[...] @ wh[...], 0.0)
    o_ref[...] = h @ w1[...] + b1[...]


def _edge_mlp(g_pair, ea_pair, wh2, w12, b12):
    nb = E2 // EB
    return pl.pallas_call(
        _t2_body,
        grid=(nb,),
        in_specs=[pl.BlockSpec((EB, 2 * D), lambda i: (i, 0)),
                  pl.BlockSpec((EB, 2 * D), lambda i: (i, 0)),
                  pl.BlockSpec((2 * D, 2 * D), lambda i: (0, 0)),
                  pl.BlockSpec((2 * D, 2 * D), lambda i: (0, 0)),
                  pl.BlockSpec((1, 2 * D), lambda i: (0, 0))],
        out_specs=pl.BlockSpec((EB, 2 * D), lambda i: (i, 0)),
        out_shape=jax.ShapeDtypeStruct((E2, 2 * D), jnp.float32),
    )(g_pair, ea_pair, wh2, w12, b12)


def _t3_body(has_x, has_proj, *refs):
    if has_x:
        sp, m, cntf, xp = refs[:4]
        refs = refs[4:]
    else:
        sp, m, cntf = refs[:3]
        refs = refs[3:]
    if has_x:
        wx = refs[0]
        refs = refs[1:]
    ws, wm, wme, bn, wn2, bn2 = refs[:6]
    refs = refs[6:]
    if has_proj:
        wr, wc, be, x_ref, ab_ref = refs
    else:
        (x_ref,) = refs
    s = sp[0] + sp[1]
    c = cntf[...]
    mean = s / c
    mm = m[...]
    mm = jnp.where(jnp.isfinite(mm), mm, 0.0)
    h = s @ ws[...] + mm @ wm[...] + mean @ wme[...] + bn[...]
    if has_x:
        h = h + xp[...] @ wx[...]
    x = jnp.maximum(h, 0.0) @ wn2[...] + bn2[...]
    x_ref[...] = x
    if has_proj:
        a = x @ wr[...] + be[...]
        b = x @ wc[...]
        ab_ref[0] = a
        ab_ref[1] = b


def _node_mlp(sp, m, cntf, xp, ws, wm, wme, bn, wn2, bn2, wx=None, proj=None):
    has_x = xp is not None
    has_proj = proj is not None
    nb = NN // NB
    full = lambda shape: pl.BlockSpec(shape, lambda i: tuple(0 for _ in shape))
    in_specs = [pl.BlockSpec((2, NB, D), lambda i: (0, i, 0)),
                pl.BlockSpec((NB, D), lambda i: (i, 0)),
                pl.BlockSpec((NB, 1), lambda i: (i, 0))]
    args = [sp, m, cntf]
    if has_x:
        in_specs.append(pl.BlockSpec((NB, D), lambda i: (i, 0)))
        args.append(xp)
        in_specs.append(full((D, D)))
        args.append(wx)
    in_specs += [full((D, D)), full((D, D)), full((D, D)), full((1, D)),
                 full((D, D)), full((1, D))]
    args += [ws, wm, wme, bn, wn2, bn2]
    out_specs = [pl.BlockSpec((NB, D), lambda i: (i, 0))]
    out_shape = [jax.ShapeDtypeStruct((NN, D), jnp.float32)]
    if has_proj:
        wr, wc, be = proj
        in_specs += [full((D, D)), full((D, D)), full((1, D))]
        args += [wr, wc, be]
        out_specs.append(pl.BlockSpec((2, NB, D), lambda i: (0, i, 0)))
        out_shape.append(jax.ShapeDtypeStruct((2, NN, D), jnp.float32))
    res = pl.pallas_call(
        functools.partial(_t3_body, has_x, has_proj),
        grid=(nb,),
        in_specs=in_specs,
        out_specs=out_specs,
        out_shape=out_shape,
    )(*args)
    return res if has_proj else (res[0], None)


@functools.partial(
    pl.kernel, mesh=_MESH, compiler_params=_CP_SC,
    out_type=(jax.ShapeDtypeStruct((NW, NG, D), jnp.float32),
              jax.ShapeDtypeStruct((NW, NG, D), jnp.float32),
              jax.ShapeDtypeStruct((NW, NG, D), jnp.float32)),
    scratch_types=[pltpu.VMEM((R, D), jnp.float32),
                   pltpu.VMEM((R,), jnp.int32),
                   pltpu.VMEM((NG, D), jnp.float32),
                   pltpu.VMEM((NG, D), jnp.float32),
                   pltpu.VMEM((NG, D), jnp.float32)],
)
def _sc_pool(x_hbm, b_hbm, sp_hbm, mp_hbm, cp_hbm, x_v, b_v, sacc, macc, cacc):
    wid = _wid()
    lo = wid * R
    zero16 = jnp.zeros((16,), jnp.float32)
    one16 = jnp.full((16,), 1.0, jnp.float32)
    ninf16 = jnp.full((16,), -jnp.inf, jnp.float32)

    def init(i, _):
        for t in range(D // 16):
            sacc[i, pl.ds(16 * t, 16)] = zero16
            macc[i, pl.ds(16 * t, 16)] = ninf16
            cacc[i, pl.ds(16 * t, 16)] = zero16
        return 0
    lax.fori_loop(0, NG, init, 0)

    @pl.when(wid < NW - 1)
    def _():
        pltpu.sync_copy(x_hbm.at[pl.ds(lo, R)], x_v)
        pltpu.sync_copy(b_hbm.at[pl.ds(pl.multiple_of(lo, 8), R)], b_v)
    @pl.when(wid == NW - 1)
    def _():
        pltpu.sync_copy(x_hbm.at[pl.ds(NN - 80, 80)], x_v.at[pl.ds(0, 80)])
        pltpu.sync_copy(b_hbm.at[pl.ds(NN - 80, 80)], b_v.at[pl.ds(0, 80)])

    hi = jnp.minimum(lo + R, NN)
    nj = (hi - lo) // 16

    def body(jj, _):
        b16 = b_v[pl.ds(jj * 16, 16)]
        for l in range(16):
            g = b16[l]
            j = jj * 16 + l
            for t in range(D // 16):
                v = x_v[j, pl.ds(16 * t, 16)]
                sacc[g, pl.ds(16 * t, 16)] += v
                macc[g, pl.ds(16 * t, 16)] = jnp.maximum(macc[g, pl.ds(16 * t, 16)], v)
                cacc[g, pl.ds(16 * t, 16)] += one16
        return 0
    lax.fori_loop(0, nj, body, 0)

    pltpu.sync_copy(sacc, sp_hbm.at[wid])
    pltpu.sync_copy(macc, mp_hbm.at[wid])
    pltpu.sync_copy(cacc, cp_hbm.at[wid])


def _t4_body(sp_ref, mp_ref, cp_ref, w0, b0, w1, b1, w2, b2, w3, b3, o_ref):
    gs = jnp.sum(sp_ref[...], axis=0)
    gm = jnp.max(mp_ref[...], axis=0)
    gc = jnp.sum(cp_ref[...], axis=0)
    gmean = gs / jnp.maximum(gc, 1.0)
    gm = jnp.where(jnp.isfinite(gm), gm, 0.0)
    out = jnp.concatenate([gs, gmean, gm], axis=1)
    h = jnp.maximum(out @ w0[...] + b0[...], 0.0)
    h = jnp.maximum(h @ w1[...] + b1[...], 0.0)
    h = jnp.maximum(h @ w2[...] + b2[...], 0.0)
    o_ref[...] = h @ w3[...] + b3[...]


def _pool_head(x3, batch, ow):
    sp, mp, cp = _sc_pool(x3, batch)
    return pl.pallas_call(
        _t4_body,
        out_shape=jax.ShapeDtypeStruct((NG, ow[6].shape[1]), jnp.float32),
    )(sp, mp, cp, *ow)


def _bd(w):
    z = jnp.zeros_like(w)
    return jnp.concatenate([jnp.concatenate([w, z], 1), jnp.concatenate([z, w], 1)], 0)


def kernel(x, edge_index, edge_attr, batch, weights):
    w = list(weights)
    row = edge_index[0]
    col = edge_index[1]

    lists, counts = _sc_filter(col)

    b0 = w[1].reshape(1, D)
    b1 = w[3].reshape(1, D)
    ea_pair = _edge_mlp0(edge_attr, w[0], b0, w[2], b1)

    xcur = None
    for i in range(3):
        ea_lin = ea_pair.reshape(NE, D)
        if i == 0:
            sp, m, cnt = _sc_scatter_cnt(lists, counts, ea_lin)
            cntf = jnp.maximum(cnt.astype(jnp.float32), 1.0).reshape(NN, 1)
        else:
            sp, m = _sc_scatter(lists, counts, ea_lin)

        nbase = 4 if i == 0 else 8 + 8 * (i - 1) + 4
        wn1, bn1, wn2, bn2 = w[nbase:nbase + 4]
        off = 0 if i == 0 else D
        wx = wn1[0:D] if i != 0 else None
        ws = wn1[off:off + D]
        wm = wn1[off + D:off + 2 * D]
        wme = wn1[off + 2 * D:off + 3 * D]

        if i < 2:
            ebase = 8 + 8 * i
            we1, be1 = w[ebase], w[ebase + 1]
            proj = (we1[0:D], we1[D:2 * D], be1.reshape(1, D))
        else:
            proj = None

        xcur, ab = _node_mlp(sp, m, cntf, xcur, ws, wm, wme,
                             bn1.reshape(1, D), wn2, bn2.reshape(1, D),
                             wx=wx, proj=proj)

        if i < 2:
            ebase = 8 + 8 * i
            we1 = w[ebase]
            g_lin = _sc_gather(row, col, ab.reshape(2 * NN, D))
            g_pair = g_lin.reshape(E2, 2 * D)
            wh2 = _bd(we1[2 * D:3 * D])
            w12 = _bd(w[ebase + 2])
            b12 = jnp.concatenate([w[ebase + 3], w[ebase + 3]]).reshape(1, 2 * D)
            ea_pair = _edge_mlp(g_pair, ea_pair, wh2, w12, b12)

    ow = [w[24], w[25].reshape(1, -1), w[26], w[27].reshape(1, -1),
          w[28], w[29].reshape(1, -1), w[30], w[31].reshape(1, -1)]
    return _pool_head(xcur, batch, ow)


# speedup vs baseline: 3.3851x; 1.1567x over previous
import functools

import jax
import jax.numpy as jnp
from jax import lax
from jax.experimental import pallas as pl
from jax.experimental.pallas import tpu as pltpu
from jax.experimental.pallas import tpu_sc as plsc

NN = 10000
NE = 320000
D = 64
NG = 64
NW = 32
E2 = NE // 2
R = 320

FL = 1024
LCAP = NE + FL + 16
CE = 8000
CHS = 448
CHG = 400

_MESH = plsc.VectorSubcoreMesh(core_axis_name="c", subcore_axis_name="s")
_CP_SC = pltpu.CompilerParams(use_tc_tiling_on_sc=False, needs_layout_passes=False)


def _wid():
    return lax.axis_index("s") * 2 + lax.axis_index("c")


@functools.partial(
    pl.kernel, mesh=_MESH, compiler_params=_CP_SC,
    out_type=(jax.ShapeDtypeStruct((NW, LCAP), jnp.int32),
              jax.ShapeDtypeStruct((NW, 8), jnp.int32)),
    scratch_types=[pltpu.VMEM((CE,), jnp.int32),
                   pltpu.VMEM((FL + 80,), jnp.int32),
                   pltpu.VMEM((16,), jnp.int32)],
)
def _sc_filter(col_hbm, list_hbm, cnt_hbm, col_v, buf_v, tmp_v):
    wid = _wid()
    lo = wid * R
    hi = jnp.minimum(lo + R, NN)
    lane = lax.broadcasted_iota(jnp.int32, (16,), 0)

    def chunk_body(ci, carry):
        pltpu.sync_copy(col_hbm.at[pl.ds(ci * CE, CE)], col_v)

        def body(jg, carry):
            off_loc, hbm_off = carry
            for u in range(4):
                jj = jg * 4 + u
                c = col_v[pl.ds(jj * 16, 16)]
                gid = ci * CE + jj * 16 + lane
                n = c - lo
                msk = jnp.logical_and(c >= lo, c < hi)
                glin = jnp.where(gid < E2, gid * 2, gid * 2 - (NE - 1))
                packed = glin + (n << 19)
                key = jnp.where(msk, lane, lane + 16)
                _, vals = plsc.sort_key_val(key, packed)
                buf_v[pl.ds(off_loc, 16)] = vals
                off_loc = off_loc + plsc.all_reduce_population_count(msk)[0]

            def flush(carry):
                off_loc, hbm_off = carry
                pltpu.sync_copy(buf_v.at[pl.ds(0, FL)],
                                list_hbm.at[wid, pl.ds(pl.multiple_of(hbm_off, FL), FL)])
                for q in range(4):
                    tail = buf_v[pl.ds(FL + 16 * q, 16)]
                    buf_v[pl.ds(16 * q, 16)] = tail
                return off_loc - FL, hbm_off + FL
            return lax.cond(off_loc >= FL, flush, lambda c: c, (off_loc, hbm_off))

        return lax.fori_loop(0, CE // 64, body, carry)

    off_loc, hbm_off = lax.fori_loop(0, NE // CE, chunk_body, (0, 0))
    pltpu.sync_copy(buf_v.at[pl.ds(0, FL)],
                    list_hbm.at[wid, pl.ds(pl.multiple_of(hbm_off, FL), FL)])
    tmp_v[...] = jnp.full((16,), hbm_off + off_loc, jnp.int32)
    pltpu.sync_copy(tmp_v.at[pl.ds(0, 8)], cnt_hbm.at[wid])


def _sc_scatter_body(with_cnt, *refs):
    if with_cnt:
        (list_hbm, cnts_hbm, ea_hbm, sp_hbm, m_hbm, c_hbm,
         pk_v, ids_v, cols_v, ea_v, macc_v, cnt_v, c8_v, acc_sh, sem0, sem1) = refs
    else:
        (list_hbm, cnts_hbm, ea_hbm, sp_hbm, m_hbm,
         pk_v, ids_v, cols_v, ea_v, macc_v, cnt_v, c8_v, acc_sh, sem0, sem1) = refs
    cid = lax.axis_index("c")
    sid = lax.axis_index("s")
    wid = sid * 2 + cid
    lo = wid * R
    lane = lax.broadcasted_iota(jnp.int32, (16,), 0)
    one0 = jnp.where(lane == 0, 1, 0)
    zero16 = jnp.zeros((16,), jnp.float32)
    ninf16 = jnp.full((16,), -jnp.inf, jnp.float32)

    def zb(i, _):
        for t in range(D // 16):
            macc_v[i, pl.ds(16 * t, 16)] = zero16
        return 0
    lax.fori_loop(0, 125, zb, 0)
    for k in range(5):
        pltpu.sync_copy(macc_v.at[pl.ds(0, 125)], acc_sh.at[pl.ds(sid * 625 + k * 125, 125)])
    @pl.when(sid == 15)
    def _():
        pltpu.sync_copy(macc_v.at[pl.ds(0, 16)], acc_sh.at[pl.ds(NN, 16)])

    def im(i, _):
        for t in range(D // 16):
            macc_v[i, pl.ds(16 * t, 16)] = ninf16
        return 0
    lax.fori_loop(0, R + 8, im, 0)
    if with_cnt:
        def ic(i, _):
            cnt_v[pl.ds(16 * i, 16)] = jnp.zeros((16,), jnp.int32)
            return 0
        lax.fori_loop(0, (R + 24) // 16, ic, 0)

    plsc.subcore_barrier()

    pltpu.sync_copy(cnts_hbm.at[wid], c8_v.at[pl.ds(0, 8)])
    count = c8_v[pl.ds(0, 16)][0]
    nch = (count + CHS - 1) // CHS

    pkb = (pk_v.at[0], pk_v.at[1])
    idb = (ids_v.at[0], ids_v.at[1])
    cob = (cols_v.at[0], cols_v.at[1])
    eab = (ea_v.at[0], ea_v.at[1])
    semb = (sem0, sem1)

    def issue(ci, b):
        base = ci * CHS
        pltpu.sync_copy(list_hbm.at[wid, pl.ds(pl.multiple_of(base, CHS), CHS)],
                        pkb[b])

        def up(jj, _):
            pk = pkb[b][pl.ds(jj * 16, 16)]
            valid = (base + jj * 16 + lane) < count
            pkb[b][pl.ds(jj * 16, 16)] = jnp.where(valid, pk, R << 19)
            idb[b][pl.ds(jj * 16, 16)] = jnp.where(valid, pk & 0x7FFFF, 0)
            cob[b][pl.ds(jj * 16, 16)] = jnp.where(valid, (pk >> 19) + lo, NN)
            return 0
        lax.fori_loop(0, CHS // 16, up, 0)
        pltpu.async_copy(ea_hbm.at[idb[b]], eab[b], semb[b])

    @pl.when(nch > 0)
    def _():
        issue(0, 0)

    def pair(pi, _):
        for b in range(2):
            ci = pi * 2 + b
            @pl.when(ci < nch)
            def _():
                pltpu.make_async_copy(ea_hbm.at[idb[b]], eab[b], semb[b]).wait()
                @pl.when(ci + 1 < nch)
                def _():
                    issue(ci + 1, 1 - b)

                def g16(jj, _):
                    pk16 = pkb[b][pl.ds(jj * 16, 16)]
                    for l in range(16):
                        n = pk16[l] >> 19
                        j = jj * 16 + l
                        for t in range(D // 16):
                            v = eab[b][j, pl.ds(16 * t, 16)]
                            macc_v[n, pl.ds(16 * t, 16)] = jnp.maximum(
                                macc_v[n, pl.ds(16 * t, 16)], v)
                        if with_cnt:
                            cnt_v[pl.ds(n, 16)] += one0
                    return 0
                lax.fori_loop(0, CHS // 16, g16, 0)
                pltpu.sync_copy(eab[b], acc_sh.at[cob[b]], add=True)
        return 0
    lax.fori_loop(0, (nch + 1) // 2, pair, 0)

    @pl.when(wid < NW - 1)
    def _():
        pltpu.sync_copy(macc_v.at[pl.ds(0, R)], m_hbm.at[pl.ds(lo, R)])
        if with_cnt:
            pltpu.sync_copy(cnt_v.at[pl.ds(0, R)],
                            c_hbm.at[pl.ds(pl.multiple_of(lo, 8), R)])
    @pl.when(wid == NW - 1)
    def _():
        pltpu.sync_copy(macc_v.at[pl.ds(0, 80)], m_hbm.at[pl.ds(NN - 80, 80)])
        if with_cnt:
            pltpu.sync_copy(cnt_v.at[pl.ds(0, 80)], c_hbm.at[pl.ds(NN - 80, 80)])

    plsc.subcore_barrier()
    @pl.when(sid == 0)
    def _():
        pltpu.sync_copy(acc_sh.at[pl.ds(0, NN)], sp_hbm.at[cid])


def _make_scatter(with_cnt):
    outs = [jax.ShapeDtypeStruct((2, NN, D), jnp.float32),
            jax.ShapeDtypeStruct((NN, D), jnp.float32)]
    if with_cnt:
        outs.append(jax.ShapeDtypeStruct((NN,), jnp.int32))
    return functools.partial(
        pl.kernel, mesh=_MESH, compiler_params=_CP_SC,
        out_type=tuple(outs),
        scratch_types=[pltpu.VMEM((2, CHS), jnp.int32),
                       pltpu.VMEM((2, CHS), jnp.int32),
                       pltpu.VMEM((2, CHS), jnp.int32),
                       pltpu.VMEM((2, CHS, D), jnp.float32),
                       pltpu.VMEM((R + 8, D), jnp.float32),
                       pltpu.VMEM((R + 24,), jnp.int32),
                       pltpu.VMEM((16,), jnp.int32),
                       pltpu.VMEM_SHARED((NN + 16, D), jnp.float32),
                       pltpu.SemaphoreType.DMA,
                       pltpu.SemaphoreType.DMA],
    )(functools.partial(_sc_scatter_body, with_cnt))


_sc_scatter_cnt = _make_scatter(True)
_sc_scatter = _make_scatter(False)


NCHG = (NE // NW) // CHG


@functools.partial(
    pl.kernel, mesh=_MESH, compiler_params=_CP_SC,
    out_type=(jax.ShapeDtypeStruct((NE, D), jnp.float32),
              jax.ShapeDtypeStruct((NE, D), jnp.float32)),
    scratch_types=[pltpu.VMEM((2, CHG), jnp.int32),
                   pltpu.VMEM((2, CHG), jnp.int32),
                   pltpu.VMEM((2, 2 * CHG), jnp.int32),
                   pltpu.VMEM((2, 2 * CHG, D), jnp.float32),
                   pltpu.SemaphoreType.DMA,
                   pltpu.SemaphoreType.DMA],
)
def _sc_gather(row_hbm, col_hbm, ab_hbm, ga_hbm, gb_hbm,
               str_v, stc_v, gl_v, rows_v, sem0, sem1):
    wid = _wid()
    lane = lax.broadcasted_iota(jnp.int32, (16,), 0)
    H = CHG // 2
    ebase = wid * (NE // NW) // 2
    strb = (str_v.at[0], str_v.at[1])
    stcb = (stc_v.at[0], stc_v.at[1])
    glb = (gl_v.at[0], gl_v.at[1])
    rob = (rows_v.at[0], rows_v.at[1])
    semb = (sem0, sem1)

    def issue(ci, b):
        ea = ebase + ci * H
        pltpu.sync_copy(row_hbm.at[pl.ds(pl.multiple_of(ea, 8), H)], strb[b].at[pl.ds(0, H)])
        pltpu.sync_copy(row_hbm.at[pl.ds(pl.multiple_of(E2 + ea, 8), H)], strb[b].at[pl.ds(H, H)])
        pltpu.sync_copy(col_hbm.at[pl.ds(pl.multiple_of(ea, 8), H)], stcb[b].at[pl.ds(0, H)])
        pltpu.sync_copy(col_hbm.at[pl.ds(pl.multiple_of(E2 + ea, 8), H)], stcb[b].at[pl.ds(H, H)])

        def up(jj, _):
            lm = jj * 16 + lane
            src = (lm >> 1) + (lm & 1) * H
            glb[b][pl.ds(jj * 16, 16)] = plsc.load_gather(strb[b], [src])
            glb[b][pl.ds(CHG + jj * 16, 16)] = plsc.load_gather(stcb[b], [src]) + NN
            return 0
        lax.fori_loop(0, CHG // 16, up, 0)
        pltpu.async_copy(ab_hbm.at[glb[b]], rob[b], semb[b])

    issue(0, 0)

    def pair(pi, _):
        for b in range(2):
            ci = pi * 2 + b
            @pl.when(ci < NCHG)
            def _():
                base = wid * (NE // NW) + ci * CHG
                pltpu.make_async_copy(ab_hbm.at[glb[b]], rob[b], semb[b]).wait()
                @pl.when(ci + 1 < NCHG)
                def _():
                    issue(ci + 1, 1 - b)
                pltpu.sync_copy(rob[b].at[pl.ds(0, CHG)], ga_hbm.at[pl.ds(base, CHG)])
                pltpu.sync_copy(rob[b].at[pl.ds(CHG, CHG)], gb_hbm.at[pl.ds(base, CHG)])
        return 0
    lax.fori_loop(0, (NCHG + 1) // 2, pair, 0)


EB = 4000
NB = 2000


def _t1_body(a_ref, b_ref, w0, b0, w1, b1, o_ref):
    ha = jnp.maximum(a_ref[...] @ w0[...] + b0[...], 0.0) @ w1[...] + b1[...]
    hb = jnp.maximum(b_ref[...] @ w0[...] + b0[...], 0.0) @ w1[...] + b1[...]
    o_ref[...] = jnp.concatenate([ha, hb], axis=1)


def _edge_mlp0(edge_attr, w0, b0, w1, b1):
    nb = E2 // EB
    return pl.pallas_call(
        _t1_body,
        grid=(nb,),
        in_specs=[pl.BlockSpec((EB, 16), lambda i: (i, 0)),
                  pl.BlockSpec((EB, 16), lambda i: (i + E2 // EB, 0)),
                  pl.BlockSpec((16, D), lambda i: (0, 0)),
                  pl.BlockSpec((1, D), lambda i: (0, 0)),
                  pl.BlockSpec((D, D), lambda i: (0, 0)),
                  pl.BlockSpec((1, D), lambda i: (0, 0))],
        out_specs=pl.BlockSpec((EB, 2 * D), lambda i: (i, 0)),
        out_shape=jax.ShapeDtypeStruct((E2, 2 * D), jnp.float32),
    )(edge_attr, edge_attr, w0, b0, w1, b1)


def _t2_body(ga_ref, gb_ref, ea_ref, wh, w1, b1, o_ref):
    h = jnp.maximum(ga_ref[...] + gb_ref[...] + ea_ref[...] @ wh[...], 0.0)
    o_ref[...] = h @ w1[...] + b1[...]


def _edge_mlp(ga_pair, gb_pair, ea_pair, wh2, w12, b12):
    nb = E2 // EB
    return pl.pallas_call(
        _t2_body,
        grid=(nb,),
        in_specs=[pl.BlockSpec((EB, 2 * D), lambda i: (i, 0)),
                  pl.BlockSpec((EB, 2 * D), lambda i: (i, 0)),
                  pl.BlockSpec((EB, 2 * D), lambda i: (i, 0)),
                  pl.BlockSpec((2 * D, 2 * D), lambda i: (0, 0)),
                  pl.BlockSpec((2 * D, 2 * D), lambda i: (0, 0)),
                  pl.BlockSpec((1, 2 * D), lambda i: (0, 0))],
        out_specs=pl.BlockSpec((EB, 2 * D), lambda i: (i, 0)),
        out_shape=jax.ShapeDtypeStruct((E2, 2 * D), jnp.float32),
    )(ga_pair, gb_pair, ea_pair, wh2, w12, b12)


def _t3_body(has_x, has_proj, *refs):
    if has_x:
        sp, m, cntf, xp = refs[:4]
        refs = refs[4:]
    else:
        sp, m, cntf = refs[:3]
        refs = refs[3:]
    if has_x:
        wx = refs[0]
        refs = refs[1:]
    ws, wm, wme, bn, wn2, bn2 = refs[:6]
    refs = refs[6:]
    if has_proj:
        wr, wc, be, x_ref, ab_ref = refs
    else:
        (x_ref,) = refs
    s = sp[0] + sp[1]
    c = cntf[...]
    mean = s / c
    mm = m[...]
    mm = jnp.where(jnp.isfinite(mm), mm, 0.0)
    h = s @ ws[...] + mm @ wm[...] + mean @ wme[...] + bn[...]
    if has_x:
        h = h + xp[...] @ wx[...]
    x = jnp.maximum(h, 0.0) @ wn2[...] + bn2[...]
    x_ref[...] = x
    if has_proj:
        a = x @ wr[...] + be[...]
        b = x @ wc[...]
        ab_ref[0] = a
        ab_ref[1] = b


def _node_mlp(sp, m, cntf, xp, ws, wm, wme, bn, wn2, bn2, wx=None, proj=None):
    has_x = xp is not None
    has_proj = proj is not None
    nb = NN // NB
    full = lambda shape: pl.BlockSpec(shape, lambda i: tuple(0 for _ in shape))
    in_specs = [pl.BlockSpec((2, NB, D), lambda i: (0, i, 0)),
                pl.BlockSpec((NB, D), lambda i: (i, 0)),
                pl.BlockSpec((NB, 1), lambda i: (i, 0))]
    args = [sp, m, cntf]
    if has_x:
        in_specs.append(pl.BlockSpec((NB, D), lambda i: (i, 0)))
        args.append(xp)
        in_specs.append(full((D, D)))
        args.append(wx)
    in_specs += [full((D, D)), full((D, D)), full((D, D)), full((1, D)),
                 full((D, D)), full((1, D))]
    args += [ws, wm, wme, bn, wn2, bn2]
    out_specs = [pl.BlockSpec((NB, D), lambda i: (i, 0))]
    out_shape = [jax.ShapeDtypeStruct((NN, D), jnp.float32)]
    if has_proj:
        wr, wc, be = proj
        in_specs += [full((D, D)), full((D, D)), full((1, D))]
        args += [wr, wc, be]
        out_specs.append(pl.BlockSpec((2, NB, D), lambda i: (0, i, 0)))
        out_shape.append(jax.ShapeDtypeStruct((2, NN, D), jnp.float32))
    res = pl.pallas_call(
        functools.partial(_t3_body, has_x, has_proj),
        grid=(nb,),
        in_specs=in_specs,
        out_specs=out_specs,
        out_shape=out_shape,
    )(*args)
    return res if has_proj else (res[0], None)


@functools.partial(
    pl.kernel, mesh=_MESH, compiler_params=_CP_SC,
    out_type=(jax.ShapeDtypeStruct((NW, NG, D), jnp.float32),
              jax.ShapeDtypeStruct((NW, NG, D), jnp.float32),
              jax.ShapeDtypeStruct((NW, NG, D), jnp.float32)),
    scratch_types=[pltpu.VMEM((R, D), jnp.float32),
                   pltpu.VMEM((R,), jnp.int32),
                   pltpu.VMEM((NG, D), jnp.float32),
                   pltpu.VMEM((NG, D), jnp.float32),
                   pltpu.VMEM((NG, D), jnp.float32)],
)
def _sc_pool(x_hbm, b_hbm, sp_hbm, mp_hbm, cp_hbm, x_v, b_v, sacc, macc, cacc):
    wid = _wid()
    lo = wid * R
    zero16 = jnp.zeros((16,), jnp.float32)
    one16 = jnp.full((16,), 1.0, jnp.float32)
    ninf16 = jnp.full((16,), -jnp.inf, jnp.float32)

    def init(i, _):
        for t in range(D // 16):
            sacc[i, pl.ds(16 * t, 16)] = zero16
            macc[i, pl.ds(16 * t, 16)] = ninf16
            cacc[i, pl.ds(16 * t, 16)] = zero16
        return 0
    lax.fori_loop(0, NG, init, 0)

    @pl.when(wid < NW - 1)
    def _():
        pltpu.sync_copy(x_hbm.at[pl.ds(lo, R)], x_v)
        pltpu.sync_copy(b_hbm.at[pl.ds(pl.multiple_of(lo, 8), R)], b_v)
    @pl.when(wid == NW - 1)
    def _():
        pltpu.sync_copy(x_hbm.at[pl.ds(NN - 80, 80)], x_v.at[pl.ds(0, 80)])
        pltpu.sync_copy(b_hbm.at[pl.ds(NN - 80, 80)], b_v.at[pl.ds(0, 80)])

    hi = jnp.minimum(lo + R, NN)
    nj = (hi - lo) // 16

    def body(jj, _):
        b16 = b_v[pl.ds(jj * 16, 16)]
        for l in range(16):
            g = b16[l]
            j = jj * 16 + l
            for t in range(D // 16):
                v = x_v[j, pl.ds(16 * t, 16)]
                sacc[g, pl.ds(16 * t, 16)] += v
                macc[g, pl.ds(16 * t, 16)] = jnp.maximum(macc[g, pl.ds(16 * t, 16)], v)
                cacc[g, pl.ds(16 * t, 16)] += one16
        return 0
    lax.fori_loop(0, nj, body, 0)

    pltpu.sync_copy(sacc, sp_hbm.at[wid])
    pltpu.sync_copy(macc, mp_hbm.at[wid])
    pltpu.sync_copy(cacc, cp_hbm.at[wid])


def _t4_body(sp_ref, mp_ref, cp_ref, w0, b0, w1, b1, w2, b2, w3, b3, o_ref):
    gs = jnp.sum(sp_ref[...], axis=0)
    gm = jnp.max(mp_ref[...], axis=0)
    gc = jnp.sum(cp_ref[...], axis=0)
    gmean = gs / jnp.maximum(gc, 1.0)
    gm = jnp.where(jnp.isfinite(gm), gm, 0.0)
    out = jnp.concatenate([gs, gmean, gm], axis=1)
    h = jnp.maximum(out @ w0[...] + b0[...], 0.0)
    h = jnp.maximum(h @ w1[...] + b1[...], 0.0)
    h = jnp.maximum(h @ w2[...] + b2[...], 0.0)
    o_ref[...] = h @ w3[...] + b3[...]


def _pool_head(x3, batch, ow):
    sp, mp, cp = _sc_pool(x3, batch)
    return pl.pallas_call(
        _t4_body,
        out_shape=jax.ShapeDtypeStruct((NG, ow[6].shape[1]), jnp.float32),
    )(sp, mp, cp, *ow)


def _bd(w):
    z = jnp.zeros_like(w)
    return jnp.concatenate([jnp.concatenate([w, z], 1), jnp.concatenate([z, w], 1)], 0)


def kernel(x, edge_index, edge_attr, batch, weights):
    w = list(weights)
    row = edge_index[0]
    col = edge_index[1]

    lists, counts = _sc_filter(col)

    b0 = w[1].reshape(1, D)
    b1 = w[3].reshape(1, D)
    ea_pair = _edge_mlp0(edge_attr, w[0], b0, w[2], b1)

    xcur = None
    for i in range(3):
        ea_lin = ea_pair.reshape(NE, D)
        if i == 0:
            sp, m, cnt = _sc_scatter_cnt(lists, counts, ea_lin)
            cntf = jnp.maximum(cnt.astype(jnp.float32), 1.0).reshape(NN, 1)
        else:
            sp, m = _sc_scatter(lists, counts, ea_lin)

        nbase = 4 if i == 0 else 8 + 8 * (i - 1) + 4
        wn1, bn1, wn2, bn2 = w[nbase:nbase + 4]
        off = 0 if i == 0 else D
        wx = wn1[0:D] if i != 0 else None
        ws = wn1[off:off + D]
        wm = wn1[off + D:off + 2 * D]
        wme = wn1[off + 2 * D:off + 3 * D]

        if i < 2:
            ebase = 8 + 8 * i
            we1, be1 = w[ebase], w[ebase + 1]
            proj = (we1[0:D], we1[D:2 * D], be1.reshape(1, D))
        else:
            proj = None

        xcur, ab = _node_mlp(sp, m, cntf, xcur, ws, wm, wme,
                             bn1.reshape(1, D), wn2, bn2.reshape(1, D),
                             wx=wx, proj=proj)

        if i < 2:
            ebase = 8 + 8 * i
            we1 = w[ebase]
            ga_lin, gb_lin = _sc_gather(row, col, ab.reshape(2 * NN, D))
            wh2 = _bd(we1[2 * D:3 * D])
            w12 = _bd(w[ebase + 2])
            b12 = jnp.concatenate([w[ebase + 3], w[ebase + 3]]).reshape(1, 2 * D)
            ea_pair = _edge_mlp(ga_lin.reshape(E2, 2 * D), gb_lin.reshape(E2, 2 * D),
                                ea_pair, wh2, w12, b12)

    ow = [w[24], w[25].reshape(1, -1), w[26], w[27].reshape(1, -1),
          w[28], w[29].reshape(1, -1), w[30], w[31].reshape(1, -1)]
    return _pool_head(xcur, batch, ow)


# speedup vs baseline: 3.4260x; 1.0121x over previous
import functools

import jax
import jax.numpy as jnp
from jax import lax
from jax.experimental import pallas as pl
from jax.experimental.pallas import tpu as pltpu
from jax.experimental.pallas import tpu_sc as plsc

NN = 10000
NE = 320000
D = 64
NG = 64
NW = 32
E2 = NE // 2
R = 320

FL = 1024
LCAP = NE + 2048
CE = 8000
CHS = 576
HN = 16 * R
CHG = 400

_MESH = plsc.VectorSubcoreMesh(core_axis_name="c", subcore_axis_name="s")
_CP_SC = pltpu.CompilerParams(use_tc_tiling_on_sc=False, needs_layout_passes=False)


def _wid():
    return lax.axis_index("s") * 2 + lax.axis_index("c")


@functools.partial(
    pl.kernel, mesh=_MESH, compiler_params=_CP_SC,
    out_type=(jax.ShapeDtypeStruct((NW, LCAP), jnp.int32),
              jax.ShapeDtypeStruct((NW, 8), jnp.int32)),
    scratch_types=[pltpu.VMEM((CE,), jnp.int32),
                   pltpu.VMEM((FL + 80,), jnp.int32),
                   pltpu.VMEM((16,), jnp.int32)],
)
def _sc_filter(col_hbm, list_hbm, cnt_hbm, col_v, buf_v, tmp_v):
    wid = _wid()
    lo = wid * R
    hi = jnp.minimum(lo + R, NN)
    lane = lax.broadcasted_iota(jnp.int32, (16,), 0)

    def chunk_body(ci, carry):
        pltpu.sync_copy(col_hbm.at[pl.ds(ci * CE, CE)], col_v)

        def body(jg, carry):
            off_loc, hbm_off = carry
            for u in range(4):
                jj = jg * 4 + u
                c = col_v[pl.ds(jj * 16, 16)]
                gid = ci * CE + jj * 16 + lane
                n = c - lo
                msk = jnp.logical_and(c >= lo, c < hi)
                glin = jnp.where(gid < E2, gid * 2, gid * 2 - (NE - 1))
                packed = glin + (n << 19)
                key = jnp.where(msk, lane, lane + 16)
                _, vals = plsc.sort_key_val(key, packed)
                buf_v[pl.ds(off_loc, 16)] = vals
                off_loc = off_loc + plsc.all_reduce_population_count(msk)[0]

            def flush(carry):
                off_loc, hbm_off = carry
                pltpu.sync_copy(buf_v.at[pl.ds(0, FL)],
                                list_hbm.at[wid, pl.ds(pl.multiple_of(hbm_off, FL), FL)])
                for q in range(4):
                    tail = buf_v[pl.ds(FL + 16 * q, 16)]
                    buf_v[pl.ds(16 * q, 16)] = tail
                return off_loc - FL, hbm_off + FL
            return lax.cond(off_loc >= FL, flush, lambda c: c, (off_loc, hbm_off))

        return lax.fori_loop(0, CE // 64, body, carry)

    off_loc, hbm_off = lax.fori_loop(0, NE // CE, chunk_body, (0, 0))
    pltpu.sync_copy(buf_v.at[pl.ds(0, FL)],
                    list_hbm.at[wid, pl.ds(pl.multiple_of(hbm_off, FL), FL)])
    tmp_v[...] = jnp.full((16,), hbm_off + off_loc, jnp.int32)
    pltpu.sync_copy(tmp_v.at[pl.ds(0, 8)], cnt_hbm.at[wid])


def _sc_scatter_body(with_cnt, *refs):
    if with_cnt:
        (list_hbm, cnts_hbm, ea_hbm, sp_hbm, m_hbm, c_hbm,
         pk_v, ids_v, cols_v, ea_v, macc_v, cnt_v, c8_v, acc_sh, sem0, sem1) = refs
    else:
        (list_hbm, cnts_hbm, ea_hbm, sp_hbm, m_hbm,
         pk_v, ids_v, cols_v, ea_v, macc_v, cnt_v, c8_v, acc_sh, sem0, sem1) = refs
    cid = lax.axis_index("c")
    sid = lax.axis_index("s")
    rid = cid * 16 + sid
    lo = rid * R
    lloc = sid * R
    lane = lax.broadcasted_iota(jnp.int32, (16,), 0)
    one0 = jnp.where(lane == 0, 1, 0)
    zero16 = jnp.zeros((16,), jnp.float32)
    ninf16 = jnp.full((16,), -jnp.inf, jnp.float32)

    def zb(i, _):
        for t in range(D // 16):
            macc_v[i, pl.ds(16 * t, 16)] = zero16
        return 0
    lax.fori_loop(0, 160, zb, 0)
    pltpu.sync_copy(macc_v.at[pl.ds(0, 160)], acc_sh.at[pl.ds(sid * R, 160)])
    pltpu.sync_copy(macc_v.at[pl.ds(0, 160)], acc_sh.at[pl.ds(sid * R + 160, 160)])
    @pl.when(sid == 15)
    def _():
        pltpu.sync_copy(macc_v.at[pl.ds(0, 16)], acc_sh.at[pl.ds(HN, 16)])

    def im(i, _):
        for t in range(D // 16):
            macc_v[i, pl.ds(16 * t, 16)] = ninf16
        return 0
    lax.fori_loop(0, R + 8, im, 0)
    if with_cnt:
        def ic(i, _):
            cnt_v[pl.ds(16 * i, 16)] = jnp.zeros((16,), jnp.int32)
            return 0
        lax.fori_loop(0, (R + 24) // 16, ic, 0)

    plsc.subcore_barrier()

    pltpu.sync_copy(cnts_hbm.at[rid], c8_v.at[pl.ds(0, 8)])
    count = c8_v[pl.ds(0, 16)][0]
    nch = (count + CHS - 1) // CHS

    pkb = (pk_v.at[0], pk_v.at[1])
    idb = (ids_v.at[0], ids_v.at[1])
    cob = (cols_v.at[0], cols_v.at[1])
    eab = (ea_v.at[0], ea_v.at[1])
    semb = (sem0, sem1)

    def issue(ci, b):
        base = ci * CHS
        pltpu.sync_copy(list_hbm.at[rid, pl.ds(pl.multiple_of(base, CHS), CHS)],
                        pkb[b])

        def up(jj, _):
            pk = pkb[b][pl.ds(jj * 16, 16)]
            valid = (base + jj * 16 + lane) < count
            pkb[b][pl.ds(jj * 16, 16)] = jnp.where(valid, pk, R << 19)
            idb[b][pl.ds(jj * 16, 16)] = jnp.where(valid, pk & 0x7FFFF, 0)
            cob[b][pl.ds(jj * 16, 16)] = jnp.where(valid, (pk >> 19) + lloc, HN)
            return 0
        lax.fori_loop(0, CHS // 16, up, 0)
        pltpu.async_copy(ea_hbm.at[idb[b]], eab[b], semb[b])

    @pl.when(nch > 0)
    def _():
        issue(0, 0)

    def pair(pi, _):
        for b in range(2):
            ci = pi * 2 + b
            @pl.when(ci < nch)
            def _():
                pltpu.make_async_copy(ea_hbm.at[idb[b]], eab[b], semb[b]).wait()
                @pl.when(ci + 1 < nch)
                def _():
                    issue(ci + 1, 1 - b)

                def g16(jj, _):
                    pk16 = pkb[b][pl.ds(jj * 16, 16)]
                    for l in range(16):
                        n = pk16[l] >> 19
                        j = jj * 16 + l
                        for t in range(D // 16):
                            v = eab[b][j, pl.ds(16 * t, 16)]
                            macc_v[n, pl.ds(16 * t, 16)] = jnp.maximum(
                                macc_v[n, pl.ds(16 * t, 16)], v)
                        if with_cnt:
                            cnt_v[pl.ds(n, 16)] += one0
                    return 0
                lax.fori_loop(0, CHS // 16, g16, 0)
                pltpu.sync_copy(eab[b], acc_sh.at[cob[b]], add=True)
        return 0
    lax.fori_loop(0, (nch + 1) // 2, pair, 0)

    @pl.when(rid < NW - 1)
    def _():
        pltpu.sync_copy(macc_v.at[pl.ds(0, R)], m_hbm.at[pl.ds(lo, R)])
        if with_cnt:
            pltpu.sync_copy(cnt_v.at[pl.ds(0, R)],
                            c_hbm.at[pl.ds(pl.multiple_of(lo, 8), R)])
    @pl.when(rid == NW - 1)
    def _():
        pltpu.sync_copy(macc_v.at[pl.ds(0, 80)], m_hbm.at[pl.ds(NN - 80, 80)])
        if with_cnt:
            pltpu.sync_copy(cnt_v.at[pl.ds(0, 80)], c_hbm.at[pl.ds(NN - 80, 80)])

    plsc.subcore_barrier()
    @pl.when(jnp.logical_and(sid == 0, cid == 0))
    def _():
        pltpu.sync_copy(acc_sh.at[pl.ds(0, HN)], sp_hbm.at[pl.ds(0, HN)])
    @pl.when(jnp.logical_and(sid == 0, cid == 1))
    def _():
        pltpu.sync_copy(acc_sh.at[pl.ds(0, NN - HN)], sp_hbm.at[pl.ds(HN, NN - HN)])


def _make_scatter(with_cnt):
    outs = [jax.ShapeDtypeStruct((NN, D), jnp.float32),
            jax.ShapeDtypeStruct((NN, D), jnp.float32)]
    if with_cnt:
        outs.append(jax.ShapeDtypeStruct((NN,), jnp.int32))
    return functools.partial(
        pl.kernel, mesh=_MESH, compiler_params=_CP_SC,
        out_type=tuple(outs),
        scratch_types=[pltpu.VMEM((2, CHS), jnp.int32),
                       pltpu.VMEM((2, CHS), jnp.int32),
                       pltpu.VMEM((2, CHS), jnp.int32),
                       pltpu.VMEM((2, CHS, D), jnp.float32),
                       pltpu.VMEM((R + 8, D), jnp.float32),
                       pltpu.VMEM((R + 24,), jnp.int32),
                       pltpu.VMEM((16,), jnp.int32),
                       pltpu.VMEM_SHARED((HN + 16, D), jnp.float32),
                       pltpu.SemaphoreType.DMA,
                       pltpu.SemaphoreType.DMA],
    )(functools.partial(_sc_scatter_body, with_cnt))


_sc_scatter_cnt = _make_scatter(True)
_sc_scatter = _make_scatter(False)


NCHG = (NE // NW) // CHG


@functools.partial(
    pl.kernel, mesh=_MESH, compiler_params=_CP_SC,
    out_type=(jax.ShapeDtypeStruct((NE, D), jnp.float32),
              jax.ShapeDtypeStruct((NE, D), jnp.float32)),
    scratch_types=[pltpu.VMEM((2, CHG), jnp.int32),
                   pltpu.VMEM((2, CHG), jnp.int32),
                   pltpu.VMEM((2, 2 * CHG), jnp.int32),
                   pltpu.VMEM((2, 2 * CHG, D), jnp.float32),
                   pltpu.SemaphoreType.DMA,
                   pltpu.SemaphoreType.DMA],
)
def _sc_gather(row_hbm, col_hbm, ab_hbm, ga_hbm, gb_hbm,
               str_v, stc_v, gl_v, rows_v, sem0, sem1):
    wid = _wid()
    lane = lax.broadcasted_iota(jnp.int32, (16,), 0)
    H = CHG // 2
    ebase = wid * (NE // NW) // 2
    strb = (str_v.at[0], str_v.at[1])
    stcb = (stc_v.at[0], stc_v.at[1])
    glb = (gl_v.at[0], gl_v.at[1])
    rob = (rows_v.at[0], rows_v.at[1])
    semb = (sem0, sem1)

    def issue(ci, b):
        ea = ebase + ci * H
        pltpu.sync_copy(row_hbm.at[pl.ds(pl.multiple_of(ea, 8), H)], strb[b].at[pl.ds(0, H)])
        pltpu.sync_copy(row_hbm.at[pl.ds(pl.multiple_of(E2 + ea, 8), H)], strb[b].at[pl.ds(H, H)])
        pltpu.sync_copy(col_hbm.at[pl.ds(pl.multiple_of(ea, 8), H)], stcb[b].at[pl.ds(0, H)])
        pltpu.sync_copy(col_hbm.at[pl.ds(pl.multiple_of(E2 + ea, 8), H)], stcb[b].at[pl.ds(H, H)])

        def up(jj, _):
            lm = jj * 16 + lane
            src = (lm >> 1) + (lm & 1) * H
            glb[b][pl.ds(jj * 16, 16)] = plsc.load_gather(strb[b], [src])
            glb[b][pl.ds(CHG + jj * 16, 16)] = plsc.load_gather(stcb[b], [src]) + NN
            return 0
        lax.fori_loop(0, CHG // 16, up, 0)
        pltpu.async_copy(ab_hbm.at[glb[b]], rob[b], semb[b])

    issue(0, 0)

    def pair(pi, _):
        for b in range(2):
            ci = pi * 2 + b
            @pl.when(ci < NCHG)
            def _():
                base = wid * (NE // NW) + ci * CHG
                pltpu.make_async_copy(ab_hbm.at[glb[b]], rob[b], semb[b]).wait()
                @pl.when(ci + 1 < NCHG)
                def _():
                    issue(ci + 1, 1 - b)
                pltpu.sync_copy(rob[b].at[pl.ds(0, CHG)], ga_hbm.at[pl.ds(base, CHG)])
                pltpu.sync_copy(rob[b].at[pl.ds(CHG, CHG)], gb_hbm.at[pl.ds(base, CHG)])
        return 0
    lax.fori_loop(0, (NCHG + 1) // 2, pair, 0)


EB = 4000
NB = 2000


def _t1_body(a_ref, b_ref, w0, b0, w1, b1, o_ref):
    ha = jnp.maximum(a_ref[...] @ w0[...] + b0[...], 0.0) @ w1[...] + b1[...]
    hb = jnp.maximum(b_ref[...] @ w0[...] + b0[...], 0.0) @ w1[...] + b1[...]
    o_ref[...] = jnp.concatenate([ha, hb], axis=1)


def _edge_mlp0(edge_attr, w0, b0, w1, b1):
    nb = E2 // EB
    return pl.pallas_call(
        _t1_body,
        grid=(nb,),
        in_specs=[pl.BlockSpec((EB, 16), lambda i: (i, 0)),
                  pl.BlockSpec((EB, 16), lambda i: (i + E2 // EB, 0)),
                  pl.BlockSpec((16, D), lambda i: (0, 0)),
                  pl.BlockSpec((1, D), lambda i: (0, 0)),
                  pl.BlockSpec((D, D), lambda i: (0, 0)),
                  pl.BlockSpec((1, D), lambda i: (0, 0))],
        out_specs=pl.BlockSpec((EB, 2 * D), lambda i: (i, 0)),
        out_shape=jax.ShapeDtypeStruct((E2, 2 * D), jnp.float32),
    )(edge_attr, edge_attr, w0, b0, w1, b1)


def _t2_body(ga_ref, gb_ref, ea_ref, wh, w1, b1, o_ref):
    h = jnp.maximum(ga_ref[...] + gb_ref[...] + ea_ref[...] @ wh[...], 0.0)
    o_ref[...] = h @ w1[...] + b1[...]


def _edge_mlp(ga_pair, gb_pair, ea_pair, wh2, w12, b12):
    nb = E2 // EB
    return pl.pallas_call(
        _t2_body,
        grid=(nb,),
        in_specs=[pl.BlockSpec((EB, 2 * D), lambda i: (i, 0)),
                  pl.BlockSpec((EB, 2 * D), lambda i: (i, 0)),
                  pl.BlockSpec((EB, 2 * D), lambda i: (i, 0)),
                  pl.BlockSpec((2 * D, 2 * D), lambda i: (0, 0)),
                  pl.BlockSpec((2 * D, 2 * D), lambda i: (0, 0)),
                  pl.BlockSpec((1, 2 * D), lambda i: (0, 0))],
        out_specs=pl.BlockSpec((EB, 2 * D), lambda i: (i, 0)),
        out_shape=jax.ShapeDtypeStruct((E2, 2 * D), jnp.float32),
    )(ga_pair, gb_pair, ea_pair, wh2, w12, b12)


def _t3_body(has_x, has_proj, *refs):
    if has_x:
        sp, m, cntf, xp = refs[:4]
        refs = refs[4:]
    else:
        sp, m, cntf = refs[:3]
        refs = refs[3:]
    if has_x:
        wx = refs[0]
        refs = refs[1:]
    ws, wm, wme, bn, wn2, bn2 = refs[:6]
    refs = refs[6:]
    if has_proj:
        wr, wc, be, x_ref, ab_ref = refs
    else:
        (x_ref,) = refs
    s = sp[...]
    c = cntf[...]
    mean = s / c
    mm = m[...]
    mm = jnp.where(jnp.isfinite(mm), mm, 0.0)
    h = s @ ws[...] + mm @ wm[...] + mean @ wme[...] + bn[...]
    if has_x:
        h = h + xp[...] @ wx[...]
    x = jnp.maximum(h, 0.0) @ wn2[...] + bn2[...]
    x_ref[...] = x
    if has_proj:
        a = x @ wr[...] + be[...]
        b = x @ wc[...]
        ab_ref[0] = a
        ab_ref[1] = b


def _node_mlp(sp, m, cntf, xp, ws, wm, wme, bn, wn2, bn2, wx=None, proj=None):
    has_x = xp is not None
    has_proj = proj is not None
    nb = NN // NB
    full = lambda shape: pl.BlockSpec(shape, lambda i: tuple(0 for _ in shape))
    in_specs = [pl.BlockSpec((NB, D), lambda i: (i, 0)),
                pl.BlockSpec((NB, D), lambda i: (i, 0)),
                pl.BlockSpec((NB, 1), lambda i: (i, 0))]
    args = [sp, m, cntf]
    if has_x:
        in_specs.append(pl.BlockSpec((NB, D), lambda i: (i, 0)))
        args.append(xp)
        in_specs.append(full((D, D)))
        args.append(wx)
    in_specs += [full((D, D)), full((D, D)), full((D, D)), full((1, D)),
                 full((D, D)), full((1, D))]
    args += [ws, wm, wme, bn, wn2, bn2]
    out_specs = [pl.BlockSpec((NB, D), lambda i: (i, 0))]
    out_shape = [jax.ShapeDtypeStruct((NN, D), jnp.float32)]
    if has_proj:
        wr, wc, be = proj
        in_specs += [full((D, D)), full((D, D)), full((1, D))]
        args += [wr, wc, be]
        out_specs.append(pl.BlockSpec((2, NB, D), lambda i: (0, i, 0)))
        out_shape.append(jax.ShapeDtypeStruct((2, NN, D), jnp.float32))
    res = pl.pallas_call(
        functools.partial(_t3_body, has_x, has_proj),
        grid=(nb,),
        in_specs=in_specs,
        out_specs=out_specs,
        out_shape=out_shape,
    )(*args)
    return res if has_proj else (res[0], None)


@functools.partial(
    pl.kernel, mesh=_MESH, compiler_params=_CP_SC,
    out_type=(jax.ShapeDtypeStruct((NW, NG, D), jnp.float32),
              jax.ShapeDtypeStruct((NW, NG, D), jnp.float32),
              jax.ShapeDtypeStruct((NW, NG, D), jnp.float32)),
    scratch_types=[pltpu.VMEM((R, D), jnp.float32),
                   pltpu.VMEM((R,), jnp.int32),
                   pltpu.VMEM((NG, D), jnp.float32),
                   pltpu.VMEM((NG, D), jnp.float32),
                   pltpu.VMEM((NG, D), jnp.float32)],
)
def _sc_pool(x_hbm, b_hbm, sp_hbm, mp_hbm, cp_hbm, x_v, b_v, sacc, macc, cacc):
    wid = _wid()
    lo = wid * R
    zero16 = jnp.zeros((16,), jnp.float32)
    one16 = jnp.full((16,), 1.0, jnp.float32)
    ninf16 = jnp.full((16,), -jnp.inf, jnp.float32)

    def init(i, _):
        for t in range(D // 16):
            sacc[i, pl.ds(16 * t, 16)] = zero16
            macc[i, pl.ds(16 * t, 16)] = ninf16
            cacc[i, pl.ds(16 * t, 16)] = zero16
        return 0
    lax.fori_loop(0, NG, init, 0)

    @pl.when(wid < NW - 1)
    def _():
        pltpu.sync_copy(x_hbm.at[pl.ds(lo, R)], x_v)
        pltpu.sync_copy(b_hbm.at[pl.ds(pl.multiple_of(lo, 8), R)], b_v)
    @pl.when(wid == NW - 1)
    def _():
        pltpu.sync_copy(x_hbm.at[pl.ds(NN - 80, 80)], x_v.at[pl.ds(0, 80)])
        pltpu.sync_copy(b_hbm.at[pl.ds(NN - 80, 80)], b_v.at[pl.ds(0, 80)])

    hi = jnp.minimum(lo + R, NN)
    nj = (hi - lo) // 16

    def body(jj, _):
        b16 = b_v[pl.ds(jj * 16, 16)]
        for l in range(16):
            g = b16[l]
            j = jj * 16 + l
            for t in range(D // 16):
                v = x_v[j, pl.ds(16 * t, 16)]
                sacc[g, pl.ds(16 * t, 16)] += v
                macc[g, pl.ds(16 * t, 16)] = jnp.maximum(macc[g, pl.ds(16 * t, 16)], v)
                cacc[g, pl.ds(16 * t, 16)] += one16
        return 0
    lax.fori_loop(0, nj, body, 0)

    pltpu.sync_copy(sacc, sp_hbm.at[wid])
    pltpu.sync_copy(macc, mp_hbm.at[wid])
    pltpu.sync_copy(cacc, cp_hbm.at[wid])


def _t4_body(sp_ref, mp_ref, cp_ref, w0, b0, w1, b1, w2, b2, w3, b3, o_ref):
    gs = jnp.sum(sp_ref[...], axis=0)
    gm = jnp.max(mp_ref[...], axis=0)
    gc = jnp.sum(cp_ref[...], axis=0)
    gmean = gs / jnp.maximum(gc, 1.0)
    gm = jnp.where(jnp.isfinite(gm), gm, 0.0)
    out = jnp.concatenate([gs, gmean, gm], axis=1)
    h = jnp.maximum(out @ w0[...] + b0[...], 0.0)
    h = jnp.maximum(h @ w1[...] + b1[...], 0.0)
    h = jnp.maximum(h @ w2[...] + b2[...], 0.0)
    o_ref[...] = h @ w3[...] + b3[...]


def _pool_head(x3, batch, ow):
    sp, mp, cp = _sc_pool(x3, batch)
    return pl.pallas_call(
        _t4_body,
        out_shape=jax.ShapeDtypeStruct((NG, ow[6].shape[1]), jnp.float32),
    )(sp, mp, cp, *ow)


def _bd(w):
    z = jnp.zeros_like(w)
    return jnp.concatenate([jnp.concatenate([w, z], 1), jnp.concatenate([z, w], 1)], 0)


def kernel(x, edge_index, edge_attr, batch, weights):
    w = list(weights)
    row = edge_index[0]
    col = edge_index[1]

    lists, counts = _sc_filter(col)

    b0 = w[1].reshape(1, D)
    b1 = w[3].reshape(1, D)
    ea_pair = _edge_mlp0(edge_attr, w[0], b0, w[2], b1)

    xcur = None
    for i in range(3):
        ea_lin = ea_pair.reshape(NE, D)
        if i == 0:
            sp, m, cnt = _sc_scatter_cnt(lists, counts, ea_lin)
            cntf = jnp.maximum(cnt.astype(jnp.float32), 1.0).reshape(NN, 1)
        else:
            sp, m = _sc_scatter(lists, counts, ea_lin)

        nbase = 4 if i == 0 else 8 + 8 * (i - 1) + 4
        wn1, bn1, wn2, bn2 = w[nbase:nbase + 4]
        off = 0 if i == 0 else D
        wx = wn1[0:D] if i != 0 else None
        ws = wn1[off:off + D]
        wm = wn1[off + D:off + 2 * D]
        wme = wn1[off + 2 * D:off + 3 * D]

        if i < 2:
            ebase = 8 + 8 * i
            we1, be1 = w[ebase], w[ebase + 1]
            proj = (we1[0:D], we1[D:2 * D], be1.reshape(1, D))
        else:
            proj = None

        xcur, ab = _node_mlp(sp, m, cntf, xcur, ws, wm, wme,
                             bn1.reshape(1, D), wn2, bn2.reshape(1, D),
                             wx=wx, proj=proj)

        if i < 2:
            ebase = 8 + 8 * i
            we1 = w[ebase]
            ga_lin, gb_lin = _sc_gather(row, col, ab.reshape(2 * NN, D))
            wh2 = _bd(we1[2 * D:3 * D])
            w12 = _bd(w[ebase + 2])
            b12 = jnp.concatenate([w[ebase + 3], w[ebase + 3]]).reshape(1, 2 * D)
            ea_pair = _edge_mlp(ga_lin.reshape(E2, 2 * D), gb_lin.reshape(E2, 2 * D),
                                ea_pair, wh2, w12, b12)

    ow = [w[24], w[25].reshape(1, -1), w[26], w[27].reshape(1, -1),
          w[28], w[29].reshape(1, -1), w[30], w[31].reshape(1, -1)]
    return _pool_head(xcur, batch, ow)
